# Initial kernel scaffold; baseline (speedup 1.0000x reference)
import jax
import jax.numpy as jnp
from jax.experimental import pallas as pl


def kernel(kdks, valid_indices):
    raise NotImplementedError("write your pallas kernel here")


# speedup vs baseline: 1.3485x; 1.3485x over previous
import functools

import jax
import jax.numpy as jnp
from jax import lax
from jax.experimental import pallas as pl
from jax.experimental.pallas import tpu as pltpu
from jax.experimental.pallas import tpu_sc as plsc

H = 1024
W = 1024
HW = H * W
B = 524288

NC = 2
NS = 16
NW = NC * NS
NPW = B // NW
NV = NPW // 16
NR = NPW // 128
NSLOT = 4


def _ones_image():
    def body(o_ref):
        o_ref[...] = jnp.ones_like(o_ref)

    return pl.pallas_call(
        body,
        out_shape=jax.ShapeDtypeStruct((HW * 3 // 1024, 1024), jnp.float32),
        grid=(24,),
        out_specs=pl.BlockSpec((HW * 3 // 1024 // 24, 1024), lambda i: (i, 0)),
    )()


_mesh = plsc.VectorSubcoreMesh(core_axis_name="c", subcore_axis_name="s")


@functools.partial(
    pl.kernel,
    out_type=(),
    mesh=_mesh,
    compiler_params=pltpu.CompilerParams(
        needs_layout_passes=False, use_tc_tiling_on_sc=False),
    scratch_types=(
        [
            pltpu.VMEM((NPW + 16,), jnp.int32),
            pltpu.VMEM((NPW,), jnp.int32),
        ]
        + [pltpu.VMEM((128,), jnp.int32) for _ in range(3 * NSLOT)]
        + [pltpu.VMEM((128,), jnp.int32) for _ in range(3 * NSLOT)]
        + [pltpu.VMEM((128,), jnp.float32) for _ in range(3 * NSLOT)]
        + [pltpu.SemaphoreType.DMA, pltpu.SemaphoreType.DMA]
    ),
)
def _sc_scatter(img1, kdks1, vi, ei, c1, *rest):
    nsl = 3 * NSLOT
    sidx = rest[0:nsl]
    didx = rest[nsl:2 * nsl]
    vals = rest[2 * nsl:3 * nsl]
    gsem = rest[3 * nsl]
    ssem = rest[3 * nsl + 1]
    wid = lax.axis_index("s") * NC + lax.axis_index("c")
    base = wid * NPW

    pltpu.sync_copy(vi.at[pl.ds(base, NPW)], ei.at[pl.ds(0, NPW)])

    @pl.when(wid == NW - 1)
    def _():
        ei[pl.ds(NPW, 16)] = jnp.full((16,), -1, jnp.int32)

    @pl.when(wid < NW - 1)
    def _():
        pltpu.sync_copy(vi.at[pl.ds(base + NPW, 16)], ei.at[pl.ds(NPW, 16)])

    iota16 = lax.iota(jnp.int32, 16)

    def cbody(j, carry):
        cnt, pos = carry
        a = ei[pl.ds(j * 16, 16)]
        b = plsc.load_gather(ei, [pos + 1])
        m = a != b
        mi = jnp.where(m, 1, 0).astype(jnp.int32)
        ranks = cnt + jnp.cumsum(mi) - mi
        plsc.store_scatter(c1, [ranks], pos, mask=m)
        cnt = cnt + jnp.max(plsc.all_reduce_population_count(m))
        return cnt, pos + 16

    ucnt, _ = lax.fori_loop(0, NV, cbody, (jnp.int32(0), iota16))

    t = jnp.maximum(ucnt, 1)

    @pl.when(ucnt > 0)
    def _():
        def dbody(g0, carry):
            for k in range(NSLOT):
                g = g0 * NSLOT + k
                for v in range(8):
                    j = g * 8 + v
                    pos = j * 16 + iota16
                    wrap = lax.rem(pos, t)
                    sel = jnp.where(pos >= ucnt, wrap, pos)
                    cpos = plsc.load_gather(c1, [sel])
                    dest = plsc.load_gather(ei, [cpos])
                    s0 = (cpos + base) * 6
                    d0 = dest * 3
                    for c in range(3):
                        sidx[3 * k + c][pl.ds(v * 16, 16)] = s0 + c
                        didx[3 * k + c][pl.ds(v * 16, 16)] = d0 + c
            gws = [
                pltpu.async_copy(kdks1.at[sidx[i]], vals[i], gsem)
                for i in range(nsl)
            ]
            for w in gws:
                w.wait()
            sws = [
                pltpu.async_copy(vals[i], img1.at[didx[i]], ssem)
                for i in range(nsl)
            ]
            for w in sws:
                w.wait()
            return carry

        lax.fori_loop(0, NR // NSLOT, dbody, 0)


def kernel(kdks, valid_indices):
    kdks1 = kdks.reshape(6 * B)
    img1 = jax.new_ref(_ones_image().reshape(3 * HW))
    _sc_scatter(img1, kdks1, valid_indices)
    return img1[...].reshape(1, H, W, 3)
